# baseline (speedup 1.0000x reference)
import functools

import jax
import jax.numpy as jnp
from jax import lax
from jax.experimental import pallas as pl
from jax.experimental.pallas import tpu as pltpu
from jax.experimental.pallas import tpu_sc as plsc

B, S, D, E, K, H = 1, 2048, 768, 8, 2, 3072
LANES = 128
NEG = -1e30

T = 128
NT = (S * K + E * (T - 1) + T - 1) // T
P = NT * T

NC, NS = 2, 16
NW = NC * NS
CHUNK = S // NW
WREP = 128

HI = jax.lax.Precision.HIGHEST


def _router_body(x_ref, wr_ref, br_ref,
                 pos0_ref, pos1_ref, wr0_ref, wr1_ref, te_ref, va_ref):
    xb = x_ref[...]
    lg = jnp.dot(xb, wr_ref[...], preferred_element_type=jnp.float32)
    lg = lg + br_ref[...]
    lane = lax.broadcasted_iota(jnp.int32, (S, LANES), 1)
    lg = jnp.where(lane < E, lg, NEG)
    m1 = jnp.max(lg, axis=1, keepdims=True)
    i1 = jnp.min(jnp.where(lg == m1, lane, LANES), axis=1, keepdims=True)
    lg2 = jnp.where(lane == i1, NEG, lg)
    m2 = jnp.max(lg2, axis=1, keepdims=True)
    i2 = jnp.min(jnp.where(lg2 == m2, lane, LANES), axis=1, keepdims=True)
    z = jnp.exp(m2 - m1)
    wa = 1.0 / (1.0 + z)
    wb = z * wa

    oh0 = (lane == i1).astype(jnp.float32)
    oh1 = (lane == i2).astype(jnp.float32)
    ohsum = oh0 + oh1

    counts = jnp.sum(ohsum, axis=0, keepdims=True)
    padded = jnp.ceil(counts / T) * T
    upper = (lax.broadcasted_iota(jnp.int32, (LANES, LANES), 0)
             <= lax.broadcasted_iota(jnp.int32, (LANES, LANES), 1)
             ).astype(jnp.float32)
    bound = jnp.dot(padded, upper, precision=HI,
                    preferred_element_type=jnp.float32)
    start = bound - padded

    lower = (lax.broadcasted_iota(jnp.int32, (T, T), 0)
             > lax.broadcasted_iota(jnp.int32, (T, T), 1)
             ).astype(jnp.float32)
    carry = jnp.zeros((1, LANES), jnp.float32)
    ranks = []
    for b in range(S // T):
        blk = lax.slice(ohsum, (b * T, 0), ((b + 1) * T, LANES))
        within = jnp.dot(lower, blk, precision=HI,
                         preferred_element_type=jnp.float32)
        ranks.append(within + carry)
        carry = carry + jnp.sum(blk, axis=0, keepdims=True)
    rank = jnp.concatenate(ranks, axis=0)

    target = start + rank
    pos0_ref[...] = jnp.sum(oh0 * target, axis=1, keepdims=True
                            ).astype(jnp.int32)
    pos1_ref[...] = jnp.sum(oh1 * target, axis=1, keepdims=True
                            ).astype(jnp.int32)
    ones16 = jnp.ones((1, WREP), jnp.float32)
    wr0_ref[...] = wa * ones16
    wr1_ref[...] = wb * ones16

    jcol = lax.broadcasted_iota(jnp.int32, (LANES, 1), 0
                                ).astype(jnp.float32) * T
    cmp = jnp.where(lane[:LANES, :] < E,
                    (jcol >= bound).astype(jnp.float32), 0.0)
    ej = jnp.sum(cmp, axis=1, keepdims=True)
    te_ref[...] = jnp.minimum(ej, float(E - 1)).astype(jnp.int32)
    va_ref[...] = (ej < float(E)).astype(jnp.int32)


_router = pl.pallas_call(
    _router_body,
    in_specs=[
        pl.BlockSpec((S, D), lambda: (0, 0)),
        pl.BlockSpec((D, LANES), lambda: (0, 0)),
        pl.BlockSpec((1, LANES), lambda: (0, 0)),
    ],
    out_specs=[
        pl.BlockSpec((S, 1), lambda: (0, 0)),
        pl.BlockSpec((S, 1), lambda: (0, 0)),
        pl.BlockSpec((S, WREP), lambda: (0, 0)),
        pl.BlockSpec((S, WREP), lambda: (0, 0)),
        pl.BlockSpec((LANES, 1), lambda: (0, 0)),
        pl.BlockSpec((LANES, 1), lambda: (0, 0)),
    ],
    out_shape=[
        jax.ShapeDtypeStruct((S, 1), jnp.int32),
        jax.ShapeDtypeStruct((S, 1), jnp.int32),
        jax.ShapeDtypeStruct((S, WREP), jnp.float32),
        jax.ShapeDtypeStruct((S, WREP), jnp.float32),
        jax.ShapeDtypeStruct((LANES, 1), jnp.int32),
        jax.ShapeDtypeStruct((LANES, 1), jnp.int32),
    ],
)


@functools.lru_cache(maxsize=1)
def _make_sc_kernels():
    mesh = plsc.VectorSubcoreMesh(core_axis_name="c", subcore_axis_name="s")

    @functools.partial(
        pl.kernel,
        out_type=[jax.ShapeDtypeStruct((P, D), jnp.float32),
                  jax.ShapeDtypeStruct((P, WREP), jnp.float32)],
        mesh=mesh,
        scratch_types=[
            pltpu.VMEM((CHUNK, D), jnp.float32),
            pltpu.VMEM((CHUNK,), jnp.int32),
            pltpu.VMEM((CHUNK,), jnp.int32),
            pltpu.VMEM((CHUNK, WREP), jnp.float32),
            pltpu.VMEM((CHUNK, WREP), jnp.float32),
            pltpu.SemaphoreType.DMA,
        ],
    )
    def _sc_scatter(x_hbm, pos0_hbm, pos1_hbm, wr0_hbm, wr1_hbm,
                    xs_hbm, ws_hbm, rows_v, idx0_v, idx1_v, wv0, wv1, sem):
        wid = lax.axis_index("s") * NC + lax.axis_index("c")
        base = wid * CHUNK
        pltpu.sync_copy(x_hbm.at[pl.ds(base, CHUNK)], rows_v)
        pltpu.sync_copy(pos0_hbm.at[pl.ds(base, CHUNK)], idx0_v)
        pltpu.sync_copy(pos1_hbm.at[pl.ds(base, CHUNK)], idx1_v)
        pltpu.sync_copy(wr0_hbm.at[pl.ds(base, CHUNK)], wv0)
        pltpu.sync_copy(wr1_hbm.at[pl.ds(base, CHUNK)], wv1)
        cp0 = pltpu.async_copy(rows_v, xs_hbm.at[idx0_v], sem)
        cp1 = pltpu.async_copy(rows_v, xs_hbm.at[idx1_v], sem)
        cp2 = pltpu.async_copy(wv0, ws_hbm.at[idx0_v], sem)
        cp3 = pltpu.async_copy(wv1, ws_hbm.at[idx1_v], sem)
        cp0.wait()
        cp1.wait()
        cp2.wait()
        cp3.wait()

    @functools.partial(
        pl.kernel,
        out_type=jax.ShapeDtypeStruct((S, D), jnp.float32),
        mesh=mesh,
        scratch_types=[
            pltpu.VMEM((CHUNK, D), jnp.float32),
            pltpu.VMEM((CHUNK, D), jnp.float32),
            pltpu.VMEM((CHUNK,), jnp.int32),
            pltpu.VMEM((CHUNK,), jnp.int32),
            pltpu.SemaphoreType.DMA,
        ],
    )
    def _sc_combine(y_hbm, pos0_hbm, pos1_hbm, out_hbm,
                    rows_a, rows_b, idx0_v, idx1_v, sem):
        wid = lax.axis_index("s") * NC + lax.axis_index("c")
        base = wid * CHUNK
        pltpu.sync_copy(pos0_hbm.at[pl.ds(base, CHUNK)], idx0_v)
        pltpu.sync_copy(pos1_hbm.at[pl.ds(base, CHUNK)], idx1_v)
        cpa = pltpu.async_copy(y_hbm.at[idx0_v], rows_a, sem)
        cpb = pltpu.async_copy(y_hbm.at[idx1_v], rows_b, sem)
        cpa.wait()
        cpb.wait()

        def _add_row(r, _):
            for g in range(D // 16):
                sl = pl.ds(g * 16, 16)
                rows_a[r, sl] = rows_a[r, sl] + rows_b[r, sl]
            return 0

        lax.fori_loop(0, CHUNK, _add_row, 0)
        pltpu.sync_copy(rows_a, out_hbm.at[pl.ds(base, CHUNK)])

    return _sc_scatter, _sc_combine


def _gmm_body(s_ref, xs_ref, w1_ref, b1_ref, w2_ref, b2_ref, ws_ref, y_ref):
    i = pl.program_id(0)

    @pl.when(s_ref[1, i] == 1)
    def _():
        xb = xs_ref[...].astype(jnp.bfloat16)
        h = jnp.dot(xb, w1_ref[0], preferred_element_type=jnp.float32)
        h = jax.nn.relu(h + b1_ref[0, 0, :])
        y = jnp.dot(h.astype(jnp.bfloat16), w2_ref[0],
                    preferred_element_type=jnp.float32)
        y = y + b2_ref[0, 0, :]
        y_ref[...] = y * ws_ref[0][:, 0:1]


_gmm = pl.pallas_call(
    _gmm_body,
    grid_spec=pltpu.PrefetchScalarGridSpec(
        num_scalar_prefetch=1,
        grid=(NT,),
        in_specs=[
            pl.BlockSpec((T, D), lambda i, sp: (i, 0)),
            pl.BlockSpec((1, D, H), lambda i, sp: (sp[0, i], 0, 0)),
            pl.BlockSpec((1, 1, H), lambda i, sp: (sp[0, i], 0, 0)),
            pl.BlockSpec((1, H, D), lambda i, sp: (sp[0, i], 0, 0)),
            pl.BlockSpec((1, 1, D), lambda i, sp: (sp[0, i], 0, 0)),
            pl.BlockSpec((1, T, WREP), lambda i, sp: (i, 0, 0)),
        ],
        out_specs=pl.BlockSpec((T, D), lambda i, sp: (i, 0)),
    ),
    out_shape=jax.ShapeDtypeStruct((P, D), jnp.float32),
    compiler_params=pltpu.CompilerParams(
        dimension_semantics=("arbitrary",)),
)


@jax.jit
def kernel(x, W_router, b_router, W1, b1, W2, b2):
    x2d = x.reshape(S, D)
    wr_pad = jnp.zeros((D, LANES), jnp.float32).at[:, :E].set(W_router)
    br_pad = jnp.zeros((1, LANES), jnp.float32).at[0, :E].set(b_router)

    pos0c, pos1c, wrep0, wrep1, tec, vac = _router(x2d, wr_pad, br_pad)
    pos0 = pos0c.reshape(S)
    pos1 = pos1c.reshape(S)
    sp = jnp.stack([tec.reshape(LANES)[:NT], vac.reshape(LANES)[:NT]])

    _sc_scatter, _sc_combine = _make_sc_kernels()
    xs, ws = _sc_scatter(x2d, pos0, pos1, wrep0, wrep1)

    b1r = b1.reshape(E, 1, H)
    b2r = b2.reshape(E, 1, D)
    y = _gmm(sp, xs, W1.astype(jnp.bfloat16), b1r,
             W2.astype(jnp.bfloat16), b2r, ws.reshape(NT, T, WREP))

    out2d = _sc_combine(y, pos0, pos1)
    return out2d.reshape(B, S, D)


# speedup vs baseline: 1.2776x; 1.2776x over previous
import functools

import jax
import jax.numpy as jnp
from jax import lax
from jax.experimental import pallas as pl
from jax.experimental.pallas import tpu as pltpu
from jax.experimental.pallas import tpu_sc as plsc

B, S, D, E, K, H = 1, 2048, 768, 8, 2, 3072
LANES = 128
NEG = -1e30

T = 128
NT = (S * K + E * (T - 1) + T - 1) // T
P = NT * T

NC, NS = 2, 16
NW = NC * NS
CHUNK = S // NW
WREP = 128

HI = jax.lax.Precision.HIGHEST


def _router_body(x_ref, wr_ref, br_ref,
                 pos0_ref, pos1_ref, wr0_ref, wr1_ref, te_ref, va_ref):
    xb = x_ref[...]
    lg = jnp.dot(xb, wr_ref[...], preferred_element_type=jnp.float32)
    lg = lg + br_ref[...]
    lane = lax.broadcasted_iota(jnp.int32, (S, LANES), 1)
    lg = jnp.where(lane < E, lg, NEG)
    m1 = jnp.max(lg, axis=1, keepdims=True)
    i1 = jnp.min(jnp.where(lg == m1, lane, LANES), axis=1, keepdims=True)
    lg2 = jnp.where(lane == i1, NEG, lg)
    m2 = jnp.max(lg2, axis=1, keepdims=True)
    i2 = jnp.min(jnp.where(lg2 == m2, lane, LANES), axis=1, keepdims=True)
    z = jnp.exp(m2 - m1)
    wa = 1.0 / (1.0 + z)
    wb = z * wa

    oh0 = (lane == i1).astype(jnp.float32)
    oh1 = (lane == i2).astype(jnp.float32)
    ohsum = oh0 + oh1

    counts = jnp.sum(ohsum, axis=0, keepdims=True)
    padded = jnp.ceil(counts / T) * T
    upper = (lax.broadcasted_iota(jnp.int32, (LANES, LANES), 0)
             <= lax.broadcasted_iota(jnp.int32, (LANES, LANES), 1)
             ).astype(jnp.float32)
    bound = jnp.dot(padded, upper, precision=HI,
                    preferred_element_type=jnp.float32)
    start = bound - padded

    lower = (lax.broadcasted_iota(jnp.int32, (T, T), 0)
             > lax.broadcasted_iota(jnp.int32, (T, T), 1)
             ).astype(jnp.float32)
    carry = jnp.zeros((1, LANES), jnp.float32)
    ranks = []
    for b in range(S // T):
        blk = lax.slice(ohsum, (b * T, 0), ((b + 1) * T, LANES))
        within = jnp.dot(lower, blk, precision=HI,
                         preferred_element_type=jnp.float32)
        ranks.append(within + carry)
        carry = carry + jnp.sum(blk, axis=0, keepdims=True)
    rank = jnp.concatenate(ranks, axis=0)

    target = start + rank
    pos0_ref[...] = jnp.sum(oh0 * target, axis=1, keepdims=True
                            ).astype(jnp.int32)
    pos1_ref[...] = jnp.sum(oh1 * target, axis=1, keepdims=True
                            ).astype(jnp.int32)
    ones16 = jnp.ones((1, WREP), jnp.float32)
    wr0_ref[...] = wa * ones16
    wr1_ref[...] = wb * ones16

    jcol = lax.broadcasted_iota(jnp.int32, (LANES, 1), 0
                                ).astype(jnp.float32) * T
    cmp = jnp.where(lane[:LANES, :] < E,
                    (jcol >= bound).astype(jnp.float32), 0.0)
    ej = jnp.sum(cmp, axis=1, keepdims=True)
    te_ref[...] = jnp.minimum(ej, float(E - 1)).astype(jnp.int32)
    va_ref[...] = (ej < float(E)).astype(jnp.int32)


_router = pl.pallas_call(
    _router_body,
    in_specs=[
        pl.BlockSpec((S, D), lambda: (0, 0)),
        pl.BlockSpec((D, LANES), lambda: (0, 0)),
        pl.BlockSpec((1, LANES), lambda: (0, 0)),
    ],
    out_specs=[
        pl.BlockSpec((S, 1), lambda: (0, 0)),
        pl.BlockSpec((S, 1), lambda: (0, 0)),
        pl.BlockSpec((S, WREP), lambda: (0, 0)),
        pl.BlockSpec((S, WREP), lambda: (0, 0)),
        pl.BlockSpec((LANES, 1), lambda: (0, 0)),
        pl.BlockSpec((LANES, 1), lambda: (0, 0)),
    ],
    out_shape=[
        jax.ShapeDtypeStruct((S, 1), jnp.int32),
        jax.ShapeDtypeStruct((S, 1), jnp.int32),
        jax.ShapeDtypeStruct((S, WREP), jnp.float32),
        jax.ShapeDtypeStruct((S, WREP), jnp.float32),
        jax.ShapeDtypeStruct((LANES, 1), jnp.int32),
        jax.ShapeDtypeStruct((LANES, 1), jnp.int32),
    ],
)


@functools.lru_cache(maxsize=1)
def _make_sc_kernels():
    mesh = plsc.VectorSubcoreMesh(core_axis_name="c", subcore_axis_name="s")

    @functools.partial(
        pl.kernel,
        out_type=[jax.ShapeDtypeStruct((P, D), jnp.float32),
                  jax.ShapeDtypeStruct((P, WREP), jnp.float32)],
        mesh=mesh,
        scratch_types=[
            pltpu.VMEM((CHUNK, D), jnp.float32),
            pltpu.VMEM((CHUNK,), jnp.int32),
            pltpu.VMEM((CHUNK,), jnp.int32),
            pltpu.VMEM((CHUNK, WREP), jnp.float32),
            pltpu.VMEM((CHUNK, WREP), jnp.float32),
            pltpu.SemaphoreType.DMA,
        ],
    )
    def _sc_scatter(x_hbm, pos0_hbm, pos1_hbm, wr0_hbm, wr1_hbm,
                    xs_hbm, ws_hbm, rows_v, idx0_v, idx1_v, wv0, wv1, sem):
        wid = lax.axis_index("s") * NC + lax.axis_index("c")
        base = wid * CHUNK
        pltpu.sync_copy(x_hbm.at[pl.ds(base, CHUNK)], rows_v)
        pltpu.sync_copy(pos0_hbm.at[pl.ds(base, CHUNK)], idx0_v)
        pltpu.sync_copy(pos1_hbm.at[pl.ds(base, CHUNK)], idx1_v)
        pltpu.sync_copy(wr0_hbm.at[pl.ds(base, CHUNK)], wv0)
        pltpu.sync_copy(wr1_hbm.at[pl.ds(base, CHUNK)], wv1)
        cp0 = pltpu.async_copy(rows_v, xs_hbm.at[idx0_v], sem)
        cp1 = pltpu.async_copy(rows_v, xs_hbm.at[idx1_v], sem)
        cp2 = pltpu.async_copy(wv0, ws_hbm.at[idx0_v], sem)
        cp3 = pltpu.async_copy(wv1, ws_hbm.at[idx1_v], sem)
        cp0.wait()
        cp1.wait()
        cp2.wait()
        cp3.wait()

    @functools.partial(
        pl.kernel,
        out_type=jax.ShapeDtypeStruct((S, D), jnp.float32),
        mesh=mesh,
        scratch_types=[
            pltpu.VMEM((CHUNK, D), jnp.float32),
            pltpu.VMEM((CHUNK, D), jnp.float32),
            pltpu.VMEM((CHUNK,), jnp.int32),
            pltpu.VMEM((CHUNK,), jnp.int32),
            pltpu.SemaphoreType.DMA,
        ],
    )
    def _sc_combine(y_hbm, pos0_hbm, pos1_hbm, out_hbm,
                    rows_a, rows_b, idx0_v, idx1_v, sem):
        wid = lax.axis_index("s") * NC + lax.axis_index("c")
        base = wid * CHUNK
        pltpu.sync_copy(pos0_hbm.at[pl.ds(base, CHUNK)], idx0_v)
        pltpu.sync_copy(pos1_hbm.at[pl.ds(base, CHUNK)], idx1_v)
        cpa = pltpu.async_copy(y_hbm.at[idx0_v], rows_a, sem)
        cpb = pltpu.async_copy(y_hbm.at[idx1_v], rows_b, sem)
        cpa.wait()
        cpb.wait()

        def _add_row(r, _):
            for g in range(D // 16):
                sl = pl.ds(g * 16, 16)
                rows_a[r, sl] = rows_a[r, sl] + rows_b[r, sl]
            return 0

        lax.fori_loop(0, CHUNK, _add_row, 0)
        pltpu.sync_copy(rows_a, out_hbm.at[pl.ds(base, CHUNK)])

    return _sc_scatter, _sc_combine


def _gmm_body(s_ref, xs_ref, w1_ref, b1_ref, w2_ref, b2_ref, ws_ref, y_ref):
    i = pl.program_id(0)

    @pl.when(s_ref[1, i] == 1)
    def _():
        xb = xs_ref[...].astype(jnp.bfloat16)
        h = jnp.dot(xb, w1_ref[0].astype(jnp.bfloat16),
                    preferred_element_type=jnp.float32)
        h = jax.nn.relu(h + b1_ref[0, 0, :])
        y = jnp.dot(h.astype(jnp.bfloat16), w2_ref[0].astype(jnp.bfloat16),
                    preferred_element_type=jnp.float32)
        y = y + b2_ref[0, 0, :]
        y_ref[...] = y * ws_ref[0][:, 0:1]


_gmm = pl.pallas_call(
    _gmm_body,
    grid_spec=pltpu.PrefetchScalarGridSpec(
        num_scalar_prefetch=1,
        grid=(NT,),
        in_specs=[
            pl.BlockSpec((T, D), lambda i, sp: (i, 0)),
            pl.BlockSpec((1, D, H), lambda i, sp: (sp[0, i], 0, 0)),
            pl.BlockSpec((1, 1, H), lambda i, sp: (sp[0, i], 0, 0)),
            pl.BlockSpec((1, H, D), lambda i, sp: (sp[0, i], 0, 0)),
            pl.BlockSpec((1, 1, D), lambda i, sp: (sp[0, i], 0, 0)),
            pl.BlockSpec((1, T, WREP), lambda i, sp: (i, 0, 0)),
        ],
        out_specs=pl.BlockSpec((T, D), lambda i, sp: (i, 0)),
    ),
    out_shape=jax.ShapeDtypeStruct((P, D), jnp.float32),
    compiler_params=pltpu.CompilerParams(
        dimension_semantics=("arbitrary",)),
)


@jax.jit
def kernel(x, W_router, b_router, W1, b1, W2, b2):
    x2d = x.reshape(S, D)
    wr_pad = jnp.zeros((D, LANES), jnp.float32).at[:, :E].set(W_router)
    br_pad = jnp.zeros((1, LANES), jnp.float32).at[0, :E].set(b_router)

    pos0c, pos1c, wrep0, wrep1, tec, vac = _router(x2d, wr_pad, br_pad)
    pos0 = pos0c.reshape(S)
    pos1 = pos1c.reshape(S)
    sp = jnp.stack([tec.reshape(LANES)[:NT], vac.reshape(LANES)[:NT]])

    _sc_scatter, _sc_combine = _make_sc_kernels()
    xs, ws = _sc_scatter(x2d, pos0, pos1, wrep0, wrep1)

    b1r = b1.reshape(E, 1, H)
    b2r = b2.reshape(E, 1, D)
    y = _gmm(sp, xs, W1, b1r, W2, b2r, ws.reshape(NT, T, WREP))

    out2d = _sc_combine(y, pos0, pos1)
    return out2d.reshape(B, S, D)


# speedup vs baseline: 1.3729x; 1.0745x over previous
import functools

import jax
import jax.numpy as jnp
from jax import lax
from jax.experimental import pallas as pl
from jax.experimental.pallas import tpu as pltpu
from jax.experimental.pallas import tpu_sc as plsc

B, S, D, E, K, H = 1, 2048, 768, 8, 2, 3072
LANES = 128
NEG = -1e30

T = 128
NT = (S * K + E * (T - 1) + T - 1) // T
P = NT * T

NC, NS = 2, 16
NW = NC * NS
CHUNK = S // NW
WREP = 128

HI = jax.lax.Precision.HIGHEST


def _router_body(x_ref, wr_ref, br_ref,
                 pos0_ref, pos1_ref, wr0_ref, wr1_ref, te_ref, va_ref):
    xb = x_ref[...]
    lg = jnp.dot(xb, wr_ref[...], preferred_element_type=jnp.float32)
    lg = lg + br_ref[...]
    lane = lax.broadcasted_iota(jnp.int32, (S, LANES), 1)
    lg = jnp.where(lane < E, lg, NEG)
    m1 = jnp.max(lg, axis=1, keepdims=True)
    i1 = jnp.min(jnp.where(lg == m1, lane, LANES), axis=1, keepdims=True)
    lg2 = jnp.where(lane == i1, NEG, lg)
    m2 = jnp.max(lg2, axis=1, keepdims=True)
    i2 = jnp.min(jnp.where(lg2 == m2, lane, LANES), axis=1, keepdims=True)
    z = jnp.exp(m2 - m1)
    wa = 1.0 / (1.0 + z)
    wb = z * wa

    oh0 = (lane == i1).astype(jnp.float32)
    oh1 = (lane == i2).astype(jnp.float32)
    ohsum = oh0 + oh1

    counts = jnp.sum(ohsum, axis=0, keepdims=True)
    padded = jnp.ceil(counts / T) * T
    upper = (lax.broadcasted_iota(jnp.int32, (LANES, LANES), 0)
             <= lax.broadcasted_iota(jnp.int32, (LANES, LANES), 1)
             ).astype(jnp.float32)
    bound = jnp.dot(padded, upper, precision=HI,
                    preferred_element_type=jnp.float32)
    start = bound - padded

    lower = (lax.broadcasted_iota(jnp.int32, (T, T), 0)
             > lax.broadcasted_iota(jnp.int32, (T, T), 1)
             ).astype(jnp.float32)
    carry = jnp.zeros((1, LANES), jnp.float32)
    ranks = []
    for b in range(S // T):
        blk = lax.slice(ohsum, (b * T, 0), ((b + 1) * T, LANES))
        within = jnp.dot(lower, blk, precision=HI,
                         preferred_element_type=jnp.float32)
        ranks.append(within + carry)
        carry = carry + jnp.sum(blk, axis=0, keepdims=True)
    rank = jnp.concatenate(ranks, axis=0)

    target = start + rank
    pos0_ref[...] = jnp.sum(oh0 * target, axis=1, keepdims=True
                            ).astype(jnp.int32)
    pos1_ref[...] = jnp.sum(oh1 * target, axis=1, keepdims=True
                            ).astype(jnp.int32)
    ones16 = jnp.ones((1, WREP), jnp.float32)
    wr0_ref[...] = wa * ones16
    wr1_ref[...] = wb * ones16

    jcol = lax.broadcasted_iota(jnp.int32, (LANES, 1), 0
                                ).astype(jnp.float32) * T
    cmp = jnp.where(lane[:LANES, :] < E,
                    (jcol >= bound).astype(jnp.float32), 0.0)
    ej = jnp.sum(cmp, axis=1, keepdims=True)
    te_ref[...] = jnp.minimum(ej, float(E - 1)).astype(jnp.int32)
    va_ref[...] = (ej < float(E)).astype(jnp.int32)


_router = pl.pallas_call(
    _router_body,
    in_specs=[
        pl.BlockSpec((S, D), lambda: (0, 0)),
        pl.BlockSpec((D, LANES), lambda: (0, 0)),
        pl.BlockSpec((1, LANES), lambda: (0, 0)),
    ],
    out_specs=[
        pl.BlockSpec((S, 1), lambda: (0, 0)),
        pl.BlockSpec((S, 1), lambda: (0, 0)),
        pl.BlockSpec((S, WREP), lambda: (0, 0)),
        pl.BlockSpec((S, WREP), lambda: (0, 0)),
        pl.BlockSpec((LANES, 1), lambda: (0, 0)),
        pl.BlockSpec((LANES, 1), lambda: (0, 0)),
    ],
    out_shape=[
        jax.ShapeDtypeStruct((S, 1), jnp.int32),
        jax.ShapeDtypeStruct((S, 1), jnp.int32),
        jax.ShapeDtypeStruct((S, WREP), jnp.float32),
        jax.ShapeDtypeStruct((S, WREP), jnp.float32),
        jax.ShapeDtypeStruct((LANES, 1), jnp.int32),
        jax.ShapeDtypeStruct((LANES, 1), jnp.int32),
    ],
)


@functools.lru_cache(maxsize=1)
def _make_sc_kernels():
    mesh = plsc.VectorSubcoreMesh(core_axis_name="c", subcore_axis_name="s")

    @functools.partial(
        pl.kernel,
        out_type=[jax.ShapeDtypeStruct((P, D), jnp.float32),
                  jax.ShapeDtypeStruct((P, WREP), jnp.float32)],
        mesh=mesh,
        scratch_types=[
            pltpu.VMEM((CHUNK, D), jnp.float32),
            pltpu.VMEM((CHUNK,), jnp.int32),
            pltpu.VMEM((CHUNK,), jnp.int32),
            pltpu.VMEM((CHUNK, WREP), jnp.float32),
            pltpu.VMEM((CHUNK, WREP), jnp.float32),
            pltpu.SemaphoreType.DMA,
        ],
    )
    def _sc_scatter(x_hbm, pos0_hbm, pos1_hbm, wr0_hbm, wr1_hbm,
                    xs_hbm, ws_hbm, rows_v, idx0_v, idx1_v, wv0, wv1, sem):
        wid = lax.axis_index("s") * NC + lax.axis_index("c")
        base = wid * CHUNK
        pltpu.sync_copy(x_hbm.at[pl.ds(base, CHUNK)], rows_v)
        pltpu.sync_copy(pos0_hbm.at[pl.ds(base, CHUNK)], idx0_v)
        pltpu.sync_copy(pos1_hbm.at[pl.ds(base, CHUNK)], idx1_v)
        pltpu.sync_copy(wr0_hbm.at[pl.ds(base, CHUNK)], wv0)
        pltpu.sync_copy(wr1_hbm.at[pl.ds(base, CHUNK)], wv1)
        cp0 = pltpu.async_copy(rows_v, xs_hbm.at[idx0_v], sem)
        cp1 = pltpu.async_copy(rows_v, xs_hbm.at[idx1_v], sem)
        cp2 = pltpu.async_copy(wv0, ws_hbm.at[idx0_v], sem)
        cp3 = pltpu.async_copy(wv1, ws_hbm.at[idx1_v], sem)
        cp0.wait()
        cp1.wait()
        cp2.wait()
        cp3.wait()

    @functools.partial(
        pl.kernel,
        out_type=jax.ShapeDtypeStruct((S, D), jnp.float32),
        mesh=mesh,
        scratch_types=[
            pltpu.VMEM((CHUNK, D), jnp.float32),
            pltpu.VMEM((CHUNK, D), jnp.float32),
            pltpu.VMEM((CHUNK,), jnp.int32),
            pltpu.VMEM((CHUNK,), jnp.int32),
            pltpu.SemaphoreType.DMA,
        ],
    )
    def _sc_combine(y_hbm, pos0_hbm, pos1_hbm, out_hbm,
                    rows_a, rows_b, idx0_v, idx1_v, sem):
        wid = lax.axis_index("s") * NC + lax.axis_index("c")
        base = wid * CHUNK
        pltpu.sync_copy(pos0_hbm.at[pl.ds(base, CHUNK)], idx0_v)
        pltpu.sync_copy(pos1_hbm.at[pl.ds(base, CHUNK)], idx1_v)
        cpa = pltpu.async_copy(y_hbm.at[idx0_v], rows_a, sem)
        cpb = pltpu.async_copy(y_hbm.at[idx1_v], rows_b, sem)
        cpa.wait()
        cpb.wait()

        def _add_row(r, _):
            for g in range(D // 16):
                sl = pl.ds(g * 16, 16)
                rows_a[r, sl] = rows_a[r, sl] + rows_b[r, sl]
            return 0

        lax.fori_loop(0, CHUNK, _add_row, 0)
        pltpu.sync_copy(rows_a, out_hbm.at[pl.ds(base, CHUNK)])

    return _sc_scatter, _sc_combine


def _gmm_body(s_ref, xs_ref, w1_hbm, b1_ref, w2_hbm, b2_ref, ws_ref, y_ref,
              w1b, w2b, sems):
    i = pl.program_id(0)
    e = s_ref[0, i]
    first = s_ref[2, i]
    pk = s_ref[3, i]
    nxt = s_ref[4, i]

    def w1cp(src_e, buf):
        return pltpu.make_async_copy(w1_hbm.at[src_e], w1b.at[buf],
                                     sems.at[buf])

    def w2cp(src_e, buf):
        return pltpu.make_async_copy(w2_hbm.at[src_e], w2b.at[buf],
                                     sems.at[buf])

    @pl.when(i == 0)
    def _():
        w1cp(e, pk).start()
        w2cp(e, pk).start()

    @pl.when((first == 1) & (s_ref[5, i] == 1))
    def _():
        w1cp(nxt, 1 - pk).start()
        w2cp(nxt, 1 - pk).start()

    @pl.when(first == 1)
    def _():
        w1cp(e, pk).wait()
        w2cp(e, pk).wait()

    @pl.when(s_ref[1, i] == 1)
    def _():
        xb = xs_ref[...].astype(jnp.bfloat16)
        h = jnp.dot(xb, w1b[pk].astype(jnp.bfloat16),
                    preferred_element_type=jnp.float32)
        h = jax.nn.relu(h + b1_ref[0, 0, :])
        y = jnp.dot(h.astype(jnp.bfloat16), w2b[pk].astype(jnp.bfloat16),
                    preferred_element_type=jnp.float32)
        y = y + b2_ref[0, 0, :]
        y_ref[...] = y * ws_ref[0][:, 0:1]


_gmm = pl.pallas_call(
    _gmm_body,
    grid_spec=pltpu.PrefetchScalarGridSpec(
        num_scalar_prefetch=1,
        grid=(NT,),
        in_specs=[
            pl.BlockSpec((T, D), lambda i, sp: (i, 0)),
            pl.BlockSpec(memory_space=pltpu.MemorySpace.HBM),
            pl.BlockSpec((1, 1, H), lambda i, sp: (sp[0, i], 0, 0)),
            pl.BlockSpec(memory_space=pltpu.MemorySpace.HBM),
            pl.BlockSpec((1, 1, D), lambda i, sp: (sp[0, i], 0, 0)),
            pl.BlockSpec((1, T, WREP), lambda i, sp: (i, 0, 0)),
        ],
        out_specs=pl.BlockSpec((T, D), lambda i, sp: (i, 0)),
        scratch_shapes=[
            pltpu.VMEM((2, D, H), jnp.float32),
            pltpu.VMEM((2, H, D), jnp.float32),
            pltpu.SemaphoreType.DMA((2,)),
        ],
    ),
    out_shape=jax.ShapeDtypeStruct((P, D), jnp.float32),
    compiler_params=pltpu.CompilerParams(
        dimension_semantics=("arbitrary",)),
)


@jax.jit
def kernel(x, W_router, b_router, W1, b1, W2, b2):
    x2d = x.reshape(S, D)
    wr_pad = jnp.zeros((D, LANES), jnp.float32).at[:, :E].set(W_router)
    br_pad = jnp.zeros((1, LANES), jnp.float32).at[0, :E].set(b_router)

    pos0c, pos1c, wrep0, wrep1, tec, vac = _router(x2d, wr_pad, br_pad)
    pos0 = pos0c.reshape(S)
    pos1 = pos1c.reshape(S)

    te_arr = tec.reshape(LANES)[:NT]
    va_arr = vac.reshape(LANES)[:NT]
    te_prev = jnp.concatenate([te_arr[:1] - 1, te_arr[:-1]])
    first = ((te_arr != te_prev) & (va_arr == 1)).astype(jnp.int32)
    parity = (jnp.cumsum(first) - 1) % 2
    idxs = jnp.arange(NT, dtype=jnp.int32)
    startpos = jnp.where(first == 1, idxs, NT)
    nextstart = jnp.concatenate([startpos[1:], jnp.array([NT], jnp.int32)])
    suffmin = jnp.flip(jax.lax.cummin(jnp.flip(nextstart)))
    has_next = (suffmin < NT).astype(jnp.int32)
    nxt_e = te_arr[jnp.clip(suffmin, 0, NT - 1)]
    sp = jnp.stack([te_arr, va_arr, first, parity, nxt_e, has_next])

    _sc_scatter, _sc_combine = _make_sc_kernels()
    xs, ws = _sc_scatter(x2d, pos0, pos1, wrep0, wrep1)

    b1r = b1.reshape(E, 1, H)
    b2r = b2.reshape(E, 1, D)
    y = _gmm(sp, xs, W1, b1r, W2, b2r, ws.reshape(NT, T, WREP))

    out2d = _sc_combine(y, pos0, pos1)
    return out2d.reshape(B, S, D)
